# Initial kernel scaffold; baseline (speedup 1.0000x reference)
import jax
import jax.numpy as jnp
from jax.experimental import pallas as pl


def kernel(inputs, embedding):
    raise NotImplementedError("write your pallas kernel here")


# speedup vs baseline: 1.4995x; 1.4995x over previous
import functools

import jax
import jax.numpy as jnp
from jax.experimental import pallas as pl

N_BATCH = 16
N_CODES = 1024
DIM = 64
N_PIX = 1024
BLK_P = 512


def _vq_body(x_ref, e_ref, ee2_ref, xs2_ref, q_ref, loss_ref):
    X = x_ref[0]
    E = e_ref[...]
    M = jax.lax.dot_general(E, X, (((1,), (0,)), ((), ())),
                            preferred_element_type=jnp.float32)
    d = (xs2_ref[0] + ee2_ref[...]) - 2.0 * M
    m = jnp.min(d, axis=0, keepdims=True)
    iota = jax.lax.broadcasted_iota(jnp.int32, d.shape, 0)
    idx = jnp.min(jnp.where(d == m, iota, N_CODES), axis=0, keepdims=True)
    oh = (iota == idx).astype(jnp.float32)
    q = jax.lax.dot_general(E, oh, (((0,), (0,)), ((), ())),
                            preferred_element_type=jnp.float32)
    q_ref[0] = q
    diff = q - X
    part = jnp.sum(jnp.sum(diff * diff, axis=1, keepdims=True),
                   axis=0, keepdims=True)

    @pl.when((pl.program_id(0) == 0) & (pl.program_id(1) == 0))
    def _init():
        loss_ref[...] = jnp.zeros_like(loss_ref)

    loss_ref[...] += part


@jax.jit
def kernel(inputs, embedding):
    x3 = inputs.reshape(N_BATCH, DIM, N_PIX)
    ee2 = jnp.sum(embedding ** 2, axis=1).reshape(N_CODES, 1)
    xs2 = jnp.sum(jnp.transpose(inputs, (0, 2, 3, 1)).reshape(-1, DIM) ** 2,
                  axis=1).reshape(N_BATCH, 1, N_PIX)

    grid = (N_BATCH, N_PIX // BLK_P)
    q, loss_sum = pl.pallas_call(
        _vq_body,
        grid=grid,
        in_specs=[
            pl.BlockSpec((1, DIM, BLK_P), lambda n, b: (n, 0, b)),
            pl.BlockSpec((N_CODES, DIM), lambda n, b: (0, 0)),
            pl.BlockSpec((N_CODES, 1), lambda n, b: (0, 0)),
            pl.BlockSpec((1, 1, BLK_P), lambda n, b: (n, 0, b)),
        ],
        out_specs=[
            pl.BlockSpec((1, DIM, BLK_P), lambda n, b: (n, 0, b)),
            pl.BlockSpec((1, 1), lambda n, b: (0, 0)),
        ],
        out_shape=[
            jax.ShapeDtypeStruct((N_BATCH, DIM, N_PIX), jnp.float32),
            jax.ShapeDtypeStruct((1, 1), jnp.float32),
        ],
    )(x3, embedding, ee2, xs2)

    n_elems = N_BATCH * DIM * N_PIX
    loss = (1.25 / n_elems) * loss_sum[0, 0]
    return loss, q.reshape(inputs.shape)
